# baseline (speedup 1.0000x reference)
import functools

import jax
import jax.numpy as jnp
import numpy as np
from jax import lax
from jax.experimental import pallas as pl
from jax.experimental.pallas import tpu as pltpu
from jax.experimental.pallas import tpu_sc as plsc

MAX_PREFIX_LEN = 512
CONTEXT_LEN = 2048
N_GENE_VALUES = 2048
N_CELL_TYPES = 200
N_DEV_STAGES = 50
N_SEXES = 2
N = 1024
G = 19264

NUM_WORKERS = 32
ROWS_PER_WORKER = N // NUM_WORKERS
LANES = 16


def _compute_random_consts():
    cpu = jax.devices("cpu")[0]
    with jax.default_device(cpu):
        rkey = jax.random.key(1)
        k_shuf, k_prefix, k_bern = jax.random.split(rkey, 3)

        u = jax.random.uniform(k_shuf, (N, G), dtype=jnp.float32)
        shuffle_idx_nc = np.argsort(np.asarray(u), axis=-1, kind="stable")[:, :CONTEXT_LEN]

        prefix_weights = MAX_PREFIX_LEN / jnp.arange(MAX_PREFIX_LEN, dtype=jnp.float32)
        prefix_weights = prefix_weights.at[0].set(1.0)
        prefix_len_n = jax.random.categorical(k_prefix, jnp.log(prefix_weights), shape=(N,))

        metadata_weights_n = prefix_len_n.astype(jnp.float32) / (MAX_PREFIX_LEN + 1)
        bern_n3 = jax.random.bernoulli(
            k_bern, jnp.broadcast_to(metadata_weights_n[:, None], (N, 3)))

    return (
        np.ascontiguousarray(shuffle_idx_nc, dtype=np.int32),
        np.asarray(prefix_len_n, dtype=np.int32),
        np.asarray(bern_n3, dtype=np.int32),
    )


_RANDOM_CONSTS = _compute_random_consts()


@functools.lru_cache(maxsize=1)
def _random_consts():
    return _RANDOM_CONSTS


_SLAB = 2
_NSLABS = ROWS_PER_WORKER // _SLAB
_C = CONTEXT_LEN


def _sc_gather_body(table_hbm, idx_hbm, outv_hbm, outid_hbm,
                    buf0, buf1, ibuf0, ibuf1, obuf0, obuf1,
                    s_in0, s_in1, s_idx0, s_idx1, s_out0, s_out1, s_id):
    wid = lax.axis_index("s") * 2 + lax.axis_index("c")
    base = wid * ROWS_PER_WORKER

    pltpu.async_copy(idx_hbm.at[pl.ds(base, ROWS_PER_WORKER)],
                     outid_hbm.at[pl.ds(base, ROWS_PER_WORKER)], s_id)

    pltpu.async_copy(table_hbm.at[pl.ds(base, _SLAB)], buf0, s_in0)
    pltpu.async_copy(idx_hbm.at[pl.ds(base, _SLAB)], ibuf0, s_idx0)

    slots = ((buf0, ibuf0, obuf0, s_in0, s_idx0, s_out0),
             (buf1, ibuf1, obuf1, s_in1, s_idx1, s_out1))

    def outer(k, carry):
        for slot in range(2):
            buf, ibuf, obuf, s_in, s_idx, s_out = slots[slot]
            nbuf, nibuf, _, ns_in, ns_idx, _ = slots[1 - slot]
            slab_row = base + (2 * k + slot) * _SLAB
            if slot == 0:
                pltpu.async_copy(table_hbm.at[pl.ds(slab_row + _SLAB, _SLAB)],
                                 nbuf, ns_in)
                pltpu.async_copy(idx_hbm.at[pl.ds(slab_row + _SLAB, _SLAB)],
                                 nibuf, ns_idx)
            else:
                @pl.when(k < _NSLABS // 2 - 1)
                def _():
                    pltpu.async_copy(
                        table_hbm.at[pl.ds(slab_row + _SLAB, _SLAB)], nbuf, ns_in)
                    pltpu.async_copy(
                        idx_hbm.at[pl.ds(slab_row + _SLAB, _SLAB)], nibuf, ns_idx)
            pltpu.make_async_copy(table_hbm.at[pl.ds(slab_row, _SLAB)], buf, s_in).wait()
            pltpu.make_async_copy(idx_hbm.at[pl.ds(slab_row, _SLAB)], ibuf, s_idx).wait()
            @pl.when(k > 0)
            def _():
                pltpu.make_async_copy(
                    obuf, outv_hbm.at[pl.ds(slab_row, _SLAB)], s_out).wait()
            for r in range(_SLAB):
                row_sel = jnp.full((LANES,), r, jnp.int32)
                for j in range(_C // LANES):
                    ids = ibuf[r, pl.ds(j * LANES, LANES)]
                    obuf[r, pl.ds(j * LANES, LANES)] = plsc.load_gather(
                        buf, [row_sel, ids])
            pltpu.async_copy(obuf, outv_hbm.at[pl.ds(slab_row, _SLAB)], s_out)
        return carry

    lax.fori_loop(0, _NSLABS // 2, outer, 0)

    last0 = base + (_NSLABS - 2) * _SLAB
    last1 = base + (_NSLABS - 1) * _SLAB
    pltpu.make_async_copy(obuf0, outv_hbm.at[pl.ds(last0, _SLAB)], s_out0).wait()
    pltpu.make_async_copy(obuf1, outv_hbm.at[pl.ds(last1, _SLAB)], s_out1).wait()
    pltpu.make_async_copy(idx_hbm.at[pl.ds(base, ROWS_PER_WORKER)],
                          outid_hbm.at[pl.ds(base, ROWS_PER_WORKER)], s_id).wait()


def _sc_gather(table, idx):
    mesh = plsc.VectorSubcoreMesh(core_axis_name="c", subcore_axis_name="s")
    fn = pl.kernel(
        _sc_gather_body,
        out_type=(
            jax.ShapeDtypeStruct((N, CONTEXT_LEN), jnp.float32),
            jax.ShapeDtypeStruct((N, CONTEXT_LEN), jnp.int32),
        ),
        mesh=mesh,
        scratch_types=[
            pltpu.VMEM((_SLAB, G), jnp.float32),
            pltpu.VMEM((_SLAB, G), jnp.float32),
            pltpu.VMEM((_SLAB, _C), jnp.int32),
            pltpu.VMEM((_SLAB, _C), jnp.int32),
            pltpu.VMEM((_SLAB, _C), jnp.float32),
            pltpu.VMEM((_SLAB, _C), jnp.float32),
            pltpu.SemaphoreType.DMA,
            pltpu.SemaphoreType.DMA,
            pltpu.SemaphoreType.DMA,
            pltpu.SemaphoreType.DMA,
            pltpu.SemaphoreType.DMA,
            pltpu.SemaphoreType.DMA,
            pltpu.SemaphoreType.DMA,
        ],
        compiler_params=pltpu.CompilerParams(needs_layout_passes=False),
    )
    return fn(table, idx)


_ROWS_PER_BLOCK = 32
_W = CONTEXT_LEN + 3


def _asm_dep_body(g_ref, pref_ref, ch0_o, glab_o):
    R = g_ref.shape[0]
    C = CONTEXT_LEN
    lane = lax.broadcasted_iota(jnp.int32, (R, C), 1)
    p = pref_ref[...]
    suffix = lane >= p
    g = g_ref[...]
    lg = jnp.log1p(g)
    ch0_o[...] = jnp.where(suffix, 0.0, lg)
    glab_o[:, :C] = jnp.clip(g, 0, N_GENE_VALUES - 1).astype(jnp.int32)
    glab_o[:, C:] = jnp.zeros((R, 3), jnp.int32)


def _make_asm_dep(interpret=False):
    R = _ROWS_PER_BLOCK
    C = CONTEXT_LEN
    return pl.pallas_call(
        _asm_dep_body,
        grid=(N // R,),
        in_specs=[pl.BlockSpec((R, C), lambda i: (i, 0)),
                  pl.BlockSpec((R, 1), lambda i: (i, 0))],
        out_specs=[pl.BlockSpec((R, C), lambda i: (i, 0)),
                   pl.BlockSpec((R, _W), lambda i: (i, 0))],
        out_shape=[
            jax.ShapeDtypeStruct((N, C), jnp.float32),
            jax.ShapeDtypeStruct((N, _W), jnp.int32),
        ],
        interpret=interpret,
    )


def _asm_body(pref_ref, umis_ref, assay_ref, susp_ref, ct_ref, ds_ref,
              sex_ref, bern_ref,
              ch1_o, ch2_o, assay_o, susp_o, prompt_o,
              ctlab_o, dslab_o, sexlab_o,
              gw_o, ctw_o, dsw_o, sexw_o):
    R = pref_ref.shape[0]
    C = CONTEXT_LEN
    lane = lax.broadcasted_iota(jnp.int32, (R, C), 1)
    p = pref_ref[...]
    suffix = lane >= p
    ch1_o[...] = suffix.astype(jnp.float32)
    ch2_o[...] = jnp.broadcast_to(jnp.log1p(umis_ref[...]), (R, C))
    assay_o[...] = jnp.broadcast_to(assay_ref[...], (R, C))
    susp_o[...] = jnp.broadcast_to(susp_ref[...], (R, C))

    bern = bern_ref[...] != 0
    ct = ct_ref[...]
    ds = ds_ref[...]
    sx = sex_ref[...]
    measured = jnp.concatenate([ct, ds, sx], axis=1) < 0
    q3 = bern & measured
    pm3 = jnp.logical_and(~bern, measured)

    prompt_o[:, :C] = lane < p
    prompt_o[:, C:] = pm3

    zero_i1 = jnp.zeros((R, 1), jnp.int32)
    ctlab_o[:, :C] = jnp.zeros((R, C), jnp.int32)
    ctlab_o[:, C:] = jnp.concatenate([jnp.maximum(ct, 0), zero_i1, zero_i1], axis=1)
    dslab_o[:, :C] = jnp.zeros((R, C), jnp.int32)
    dslab_o[:, C:] = jnp.concatenate([zero_i1, jnp.maximum(ds, 0), zero_i1], axis=1)
    sexlab_o[:, :C] = jnp.zeros((R, C), jnp.int32)
    sexlab_o[:, C:] = jnp.concatenate([zero_i1, zero_i1, jnp.maximum(sx, 0)], axis=1)

    s = (C - p).astype(jnp.float32)
    winv = 1.0 / s
    gw_o[:, :C] = jnp.where(suffix, winv, 0.0)
    gw_o[:, C:] = jnp.zeros((R, 3), jnp.float32)
    zero_f1 = jnp.zeros((R, 1), jnp.float32)
    q3f = q3.astype(jnp.float32)
    ctw_o[:, :C] = jnp.zeros((R, C), jnp.float32)
    ctw_o[:, C:] = jnp.concatenate([q3f[:, 0:1], zero_f1, zero_f1], axis=1)
    dsw_o[:, :C] = jnp.zeros((R, C), jnp.float32)
    dsw_o[:, C:] = jnp.concatenate([zero_f1, q3f[:, 1:2], zero_f1], axis=1)
    sexw_o[:, :C] = jnp.zeros((R, C), jnp.float32)
    sexw_o[:, C:] = jnp.concatenate([zero_f1, zero_f1, q3f[:, 2:3]], axis=1)


def _make_asm(interpret=False):
    R = _ROWS_PER_BLOCK
    C = CONTEXT_LEN
    row_spec_c = pl.BlockSpec((R, C), lambda i: (i, 0))
    row_spec_w = pl.BlockSpec((R, _W), lambda i: (i, 0))
    col1 = pl.BlockSpec((R, 1), lambda i: (i, 0))
    col3 = pl.BlockSpec((R, 3), lambda i: (i, 0))
    return pl.pallas_call(
        _asm_body,
        grid=(N // R,),
        in_specs=[col1, col1, col1, col1, col1, col1, col1, col3],
        out_specs=[row_spec_c, row_spec_c, row_spec_c, row_spec_c,
                   row_spec_w,
                   row_spec_w, row_spec_w, row_spec_w,
                   row_spec_w, row_spec_w, row_spec_w, row_spec_w],
        out_shape=[
            jax.ShapeDtypeStruct((N, C), jnp.float32),
            jax.ShapeDtypeStruct((N, C), jnp.float32),
            jax.ShapeDtypeStruct((N, C), jnp.int32),
            jax.ShapeDtypeStruct((N, C), jnp.int32),
            jax.ShapeDtypeStruct((N, _W), jnp.bool_),
            jax.ShapeDtypeStruct((N, _W), jnp.int32),
            jax.ShapeDtypeStruct((N, _W), jnp.int32),
            jax.ShapeDtypeStruct((N, _W), jnp.int32),
            jax.ShapeDtypeStruct((N, _W), jnp.float32),
            jax.ShapeDtypeStruct((N, _W), jnp.float32),
            jax.ShapeDtypeStruct((N, _W), jnp.float32),
            jax.ShapeDtypeStruct((N, _W), jnp.float32),
        ],
        interpret=interpret,
    )


def kernel(cell_type_n, sex_n, development_stage_n, gene_value_ng,
           total_mrna_umis_n, assay_n, suspension_type_n):
    idx_np, prefix_np, bern_np = _random_consts()
    idx_c = jnp.asarray(idx_np)
    prefix_c = jnp.asarray(prefix_np).reshape(N, 1)
    bern_c = jnp.asarray(bern_np)

    gathered, gene_id_nc = _sc_gather(gene_value_ng, idx_c)

    (ch1, ch2, assay_nc, susp_nc, prompt_mask,
     ctlab, dslab, sexlab, gw, ctw, dsw, sexw) = _make_asm()(
        prefix_c,
        total_mrna_umis_n.reshape(N, 1).astype(jnp.float32),
        assay_n.reshape(N, 1).astype(jnp.int32),
        suspension_type_n.reshape(N, 1).astype(jnp.int32),
        cell_type_n.reshape(N, 1).astype(jnp.int32),
        development_stage_n.reshape(N, 1).astype(jnp.int32),
        sex_n.reshape(N, 1).astype(jnp.int32),
        bern_c,
    )

    ch0, glab = _make_asm_dep()(gathered, prefix_c)

    gene_value_nc3 = jnp.stack([ch0, ch1, ch2], axis=2)

    measured = jnp.stack([cell_type_n < 0, development_stage_n < 0, sex_n < 0], axis=1)
    q3 = (bern_c != 0) & measured
    cell_type_tok = jnp.where(q3[:, 0], N_CELL_TYPES,
                              jnp.maximum(cell_type_n, 0)).astype(jnp.int32)
    development_stage_tok = jnp.where(q3[:, 1], N_DEV_STAGES,
                                      jnp.maximum(development_stage_n, 0)).astype(jnp.int32)
    sex_tok = jnp.where(q3[:, 2], N_SEXES,
                        jnp.maximum(sex_n, 0)).astype(jnp.int32)

    return (
        gene_id_nc,
        gene_value_nc3,
        assay_nc,
        susp_nc,
        cell_type_tok,
        sex_tok,
        development_stage_tok,
        prompt_mask,
        glab,
        ctlab,
        dslab,
        sexlab,
        gw,
        ctw,
        dsw,
        sexw,
    )


# speedup vs baseline: 1.3669x; 1.3669x over previous
import functools

import jax
import jax.numpy as jnp
import numpy as np
from jax import lax
from jax.experimental import pallas as pl
from jax.experimental.pallas import tpu as pltpu
from jax.experimental.pallas import tpu_sc as plsc

MAX_PREFIX_LEN = 512
CONTEXT_LEN = 2048
N_GENE_VALUES = 2048
N_CELL_TYPES = 200
N_DEV_STAGES = 50
N_SEXES = 2
N = 1024
G = 19264

NUM_WORKERS = 32
ROWS_PER_WORKER = N // NUM_WORKERS
LANES = 16


def _compute_random_consts():
    cpu = jax.devices("cpu")[0]
    with jax.default_device(cpu):
        rkey = jax.random.key(1)
        k_shuf, k_prefix, k_bern = jax.random.split(rkey, 3)

        u = jax.random.uniform(k_shuf, (N, G), dtype=jnp.float32)
        shuffle_idx_nc = np.argsort(np.asarray(u), axis=-1, kind="stable")[:, :CONTEXT_LEN]

        prefix_weights = MAX_PREFIX_LEN / jnp.arange(MAX_PREFIX_LEN, dtype=jnp.float32)
        prefix_weights = prefix_weights.at[0].set(1.0)
        prefix_len_n = jax.random.categorical(k_prefix, jnp.log(prefix_weights), shape=(N,))

        metadata_weights_n = prefix_len_n.astype(jnp.float32) / (MAX_PREFIX_LEN + 1)
        bern_n3 = jax.random.bernoulli(
            k_bern, jnp.broadcast_to(metadata_weights_n[:, None], (N, 3)))

    return (
        np.ascontiguousarray(shuffle_idx_nc, dtype=np.int32),
        np.asarray(prefix_len_n, dtype=np.int32),
        np.asarray(bern_n3, dtype=np.int32),
    )


_RANDOM_CONSTS = _compute_random_consts()


@functools.lru_cache(maxsize=1)
def _random_consts():
    return _RANDOM_CONSTS


_SLAB = 2
_NSLABS = ROWS_PER_WORKER // _SLAB
_C = CONTEXT_LEN


def _sc_gather_body(table_hbm, idx_hbm, outv_hbm,
                    buf0, buf1, ibuf0, ibuf1, obuf0, obuf1,
                    s_in0, s_in1, s_idx0, s_idx1, s_out0, s_out1):
    wid = lax.axis_index("s") * 2 + lax.axis_index("c")
    base = wid * ROWS_PER_WORKER

    pltpu.async_copy(table_hbm.at[pl.ds(base, _SLAB)], buf0, s_in0)
    pltpu.async_copy(idx_hbm.at[pl.ds(base, _SLAB)], ibuf0, s_idx0)

    slots = ((buf0, ibuf0, obuf0, s_in0, s_idx0, s_out0),
             (buf1, ibuf1, obuf1, s_in1, s_idx1, s_out1))

    def outer(k, carry):
        for slot in range(2):
            buf, ibuf, obuf, s_in, s_idx, s_out = slots[slot]
            nbuf, nibuf, _, ns_in, ns_idx, _ = slots[1 - slot]
            slab_row = base + (2 * k + slot) * _SLAB
            if slot == 0:
                pltpu.async_copy(table_hbm.at[pl.ds(slab_row + _SLAB, _SLAB)],
                                 nbuf, ns_in)
                pltpu.async_copy(idx_hbm.at[pl.ds(slab_row + _SLAB, _SLAB)],
                                 nibuf, ns_idx)
            else:
                @pl.when(k < _NSLABS // 2 - 1)
                def _():
                    pltpu.async_copy(
                        table_hbm.at[pl.ds(slab_row + _SLAB, _SLAB)], nbuf, ns_in)
                    pltpu.async_copy(
                        idx_hbm.at[pl.ds(slab_row + _SLAB, _SLAB)], nibuf, ns_idx)
            pltpu.make_async_copy(table_hbm.at[pl.ds(slab_row, _SLAB)], buf, s_in).wait()
            pltpu.make_async_copy(idx_hbm.at[pl.ds(slab_row, _SLAB)], ibuf, s_idx).wait()
            @pl.when(k > 0)
            def _():
                pltpu.make_async_copy(
                    obuf, outv_hbm.at[pl.ds(slab_row, _SLAB)], s_out).wait()
            for r in range(_SLAB):
                row_sel = jnp.full((LANES,), r, jnp.int32)
                for j in range(_C // LANES):
                    ids = ibuf[r, pl.ds(j * LANES, LANES)]
                    obuf[r, pl.ds(j * LANES, LANES)] = plsc.load_gather(
                        buf, [row_sel, ids])
            pltpu.async_copy(obuf, outv_hbm.at[pl.ds(slab_row, _SLAB)], s_out)
        return carry

    lax.fori_loop(0, _NSLABS // 2, outer, 0)

    last0 = base + (_NSLABS - 2) * _SLAB
    last1 = base + (_NSLABS - 1) * _SLAB
    pltpu.make_async_copy(obuf0, outv_hbm.at[pl.ds(last0, _SLAB)], s_out0).wait()
    pltpu.make_async_copy(obuf1, outv_hbm.at[pl.ds(last1, _SLAB)], s_out1).wait()


def _sc_gather(table, idx):
    mesh = plsc.VectorSubcoreMesh(core_axis_name="c", subcore_axis_name="s")
    fn = pl.kernel(
        _sc_gather_body,
        out_type=jax.ShapeDtypeStruct((N, CONTEXT_LEN), jnp.float32),
        mesh=mesh,
        scratch_types=[
            pltpu.VMEM((_SLAB, G), jnp.float32),
            pltpu.VMEM((_SLAB, G), jnp.float32),
            pltpu.VMEM((_SLAB, _C), jnp.int32),
            pltpu.VMEM((_SLAB, _C), jnp.int32),
            pltpu.VMEM((_SLAB, _C), jnp.float32),
            pltpu.VMEM((_SLAB, _C), jnp.float32),
            pltpu.SemaphoreType.DMA,
            pltpu.SemaphoreType.DMA,
            pltpu.SemaphoreType.DMA,
            pltpu.SemaphoreType.DMA,
            pltpu.SemaphoreType.DMA,
            pltpu.SemaphoreType.DMA,
        ],
        compiler_params=pltpu.CompilerParams(needs_layout_passes=False),
    )
    return fn(table, idx)


_ROWS_PER_BLOCK = 32
_W = CONTEXT_LEN + 3


def _asm_dep_body(g_ref, pref_ref, ch0_o, glab_o):
    R = g_ref.shape[0]
    C = CONTEXT_LEN
    lane = lax.broadcasted_iota(jnp.int32, (R, C), 1)
    p = pref_ref[...]
    suffix = lane >= p
    g = g_ref[...]
    lg = jnp.log1p(g)
    ch0_o[...] = jnp.where(suffix, 0.0, lg)
    glab_o[:, :C] = jnp.clip(g, 0, N_GENE_VALUES - 1).astype(jnp.int32)
    glab_o[:, C:] = jnp.zeros((R, 3), jnp.int32)


def _make_asm_dep(interpret=False):
    R = _ROWS_PER_BLOCK
    C = CONTEXT_LEN
    return pl.pallas_call(
        _asm_dep_body,
        grid=(N // R,),
        in_specs=[pl.BlockSpec((R, C), lambda i: (i, 0)),
                  pl.BlockSpec((R, 1), lambda i: (i, 0))],
        out_specs=[pl.BlockSpec((R, C), lambda i: (i, 0)),
                   pl.BlockSpec((R, _W), lambda i: (i, 0))],
        out_shape=[
            jax.ShapeDtypeStruct((N, C), jnp.float32),
            jax.ShapeDtypeStruct((N, _W), jnp.int32),
        ],
        interpret=interpret,
    )


def _asm_body(pref_ref, umis_ref, assay_ref, susp_ref, ct_ref, ds_ref,
              sex_ref, bern_ref,
              ch1_o, ch2_o, assay_o, susp_o, prompt_o,
              ctlab_o, dslab_o, sexlab_o,
              gw_o, ctw_o, dsw_o, sexw_o):
    R = pref_ref.shape[0]
    C = CONTEXT_LEN
    lane = lax.broadcasted_iota(jnp.int32, (R, C), 1)
    p = pref_ref[...]
    suffix = lane >= p
    ch1_o[...] = suffix.astype(jnp.float32)
    ch2_o[...] = jnp.broadcast_to(jnp.log1p(umis_ref[...]), (R, C))
    assay_o[...] = jnp.broadcast_to(assay_ref[...], (R, C))
    susp_o[...] = jnp.broadcast_to(susp_ref[...], (R, C))

    bern = bern_ref[...] != 0
    ct = ct_ref[...]
    ds = ds_ref[...]
    sx = sex_ref[...]
    measured = jnp.concatenate([ct, ds, sx], axis=1) < 0
    q3 = bern & measured
    pm3 = jnp.logical_and(~bern, measured)

    prompt_o[:, :C] = lane < p
    prompt_o[:, C:] = pm3

    zero_i1 = jnp.zeros((R, 1), jnp.int32)
    ctlab_o[:, :C] = jnp.zeros((R, C), jnp.int32)
    ctlab_o[:, C:] = jnp.concatenate([jnp.maximum(ct, 0), zero_i1, zero_i1], axis=1)
    dslab_o[:, :C] = jnp.zeros((R, C), jnp.int32)
    dslab_o[:, C:] = jnp.concatenate([zero_i1, jnp.maximum(ds, 0), zero_i1], axis=1)
    sexlab_o[:, :C] = jnp.zeros((R, C), jnp.int32)
    sexlab_o[:, C:] = jnp.concatenate([zero_i1, zero_i1, jnp.maximum(sx, 0)], axis=1)

    s = (C - p).astype(jnp.float32)
    winv = 1.0 / s
    gw_o[:, :C] = jnp.where(suffix, winv, 0.0)
    gw_o[:, C:] = jnp.zeros((R, 3), jnp.float32)
    zero_f1 = jnp.zeros((R, 1), jnp.float32)
    q3f = q3.astype(jnp.float32)
    ctw_o[:, :C] = jnp.zeros((R, C), jnp.float32)
    ctw_o[:, C:] = jnp.concatenate([q3f[:, 0:1], zero_f1, zero_f1], axis=1)
    dsw_o[:, :C] = jnp.zeros((R, C), jnp.float32)
    dsw_o[:, C:] = jnp.concatenate([zero_f1, q3f[:, 1:2], zero_f1], axis=1)
    sexw_o[:, :C] = jnp.zeros((R, C), jnp.float32)
    sexw_o[:, C:] = jnp.concatenate([zero_f1, zero_f1, q3f[:, 2:3]], axis=1)


def _make_asm(interpret=False):
    R = _ROWS_PER_BLOCK
    C = CONTEXT_LEN
    row_spec_c = pl.BlockSpec((R, C), lambda i: (i, 0))
    row_spec_w = pl.BlockSpec((R, _W), lambda i: (i, 0))
    col1 = pl.BlockSpec((R, 1), lambda i: (i, 0))
    col3 = pl.BlockSpec((R, 3), lambda i: (i, 0))
    return pl.pallas_call(
        _asm_body,
        grid=(N // R,),
        in_specs=[col1, col1, col1, col1, col1, col1, col1, col3],
        out_specs=[row_spec_c, row_spec_c, row_spec_c, row_spec_c,
                   row_spec_w,
                   row_spec_w, row_spec_w, row_spec_w,
                   row_spec_w, row_spec_w, row_spec_w, row_spec_w],
        out_shape=[
            jax.ShapeDtypeStruct((N, C), jnp.float32),
            jax.ShapeDtypeStruct((N, C), jnp.float32),
            jax.ShapeDtypeStruct((N, C), jnp.int32),
            jax.ShapeDtypeStruct((N, C), jnp.int32),
            jax.ShapeDtypeStruct((N, _W), jnp.bool_),
            jax.ShapeDtypeStruct((N, _W), jnp.int32),
            jax.ShapeDtypeStruct((N, _W), jnp.int32),
            jax.ShapeDtypeStruct((N, _W), jnp.int32),
            jax.ShapeDtypeStruct((N, _W), jnp.float32),
            jax.ShapeDtypeStruct((N, _W), jnp.float32),
            jax.ShapeDtypeStruct((N, _W), jnp.float32),
            jax.ShapeDtypeStruct((N, _W), jnp.float32),
        ],
        interpret=interpret,
    )


def kernel(cell_type_n, sex_n, development_stage_n, gene_value_ng,
           total_mrna_umis_n, assay_n, suspension_type_n):
    idx_np, prefix_np, bern_np = _random_consts()
    idx_c = jnp.asarray(idx_np)
    prefix_c = jnp.asarray(prefix_np).reshape(N, 1)
    bern_c = jnp.asarray(bern_np)

    gathered = _sc_gather(gene_value_ng, idx_c)
    gene_id_nc = idx_c

    (ch1, ch2, assay_nc, susp_nc, prompt_mask,
     ctlab, dslab, sexlab, gw, ctw, dsw, sexw) = _make_asm()(
        prefix_c,
        total_mrna_umis_n.reshape(N, 1).astype(jnp.float32),
        assay_n.reshape(N, 1).astype(jnp.int32),
        suspension_type_n.reshape(N, 1).astype(jnp.int32),
        cell_type_n.reshape(N, 1).astype(jnp.int32),
        development_stage_n.reshape(N, 1).astype(jnp.int32),
        sex_n.reshape(N, 1).astype(jnp.int32),
        bern_c,
    )

    ch0, glab = _make_asm_dep()(gathered, prefix_c)

    gene_value_nc3 = jnp.stack([ch0, ch1, ch2], axis=2)

    measured = jnp.stack([cell_type_n < 0, development_stage_n < 0, sex_n < 0], axis=1)
    q3 = (bern_c != 0) & measured
    cell_type_tok = jnp.where(q3[:, 0], N_CELL_TYPES,
                              jnp.maximum(cell_type_n, 0)).astype(jnp.int32)
    development_stage_tok = jnp.where(q3[:, 1], N_DEV_STAGES,
                                      jnp.maximum(development_stage_n, 0)).astype(jnp.int32)
    sex_tok = jnp.where(q3[:, 2], N_SEXES,
                        jnp.maximum(sex_n, 0)).astype(jnp.int32)

    return (
        gene_id_nc,
        gene_value_nc3,
        assay_nc,
        susp_nc,
        cell_type_tok,
        sex_tok,
        development_stage_tok,
        prompt_mask,
        glab,
        ctlab,
        dslab,
        sexlab,
        gw,
        ctw,
        dsw,
        sexw,
    )


# speedup vs baseline: 1.8122x; 1.3258x over previous
import functools

import jax
import jax.numpy as jnp
import numpy as np
from jax import lax
from jax.experimental import pallas as pl
from jax.experimental.pallas import tpu as pltpu
from jax.experimental.pallas import tpu_sc as plsc

MAX_PREFIX_LEN = 512
CONTEXT_LEN = 2048
N_GENE_VALUES = 2048
N_CELL_TYPES = 200
N_DEV_STAGES = 50
N_SEXES = 2
N = 1024
G = 19264

NUM_WORKERS = 32
ROWS_PER_WORKER = N // NUM_WORKERS
LANES = 16


def _rotl(x, r):
    return ((x << np.uint32(r)) | (x >> np.uint32(32 - r))).astype(np.uint32)


def _threefry_pair(k1, k2, x0, x1):
    x0 = x0.astype(np.uint32).copy()
    x1 = x1.astype(np.uint32).copy()
    ks = [np.uint32(k1), np.uint32(k2),
          np.uint32(np.uint32(k1) ^ np.uint32(k2) ^ np.uint32(0x1BD11BDA))]
    rotations = [[13, 15, 26, 6], [17, 29, 16, 24]]
    with np.errstate(over="ignore"):
        x0 = (x0 + ks[0]).astype(np.uint32)
        x1 = (x1 + ks[1]).astype(np.uint32)
        for i in range(5):
            for r in rotations[i % 2]:
                x0 = (x0 + x1).astype(np.uint32)
                x1 = _rotl(x1, r) ^ x0
            x0 = (x0 + ks[(i + 1) % 3]).astype(np.uint32)
            x1 = (x1 + ks[(i + 2) % 3] + np.uint32(i + 1)).astype(np.uint32)
    return x0, x1


def _iota_2x32(n):
    i = np.arange(n, dtype=np.uint64)
    return ((i >> np.uint64(32)).astype(np.uint32),
            (i & np.uint64(0xFFFFFFFF)).astype(np.uint32))


def _np_random_bits(k1, k2, n):
    hi, lo = _iota_2x32(n)
    b1, b2 = _threefry_pair(k1, k2, hi, lo)
    return b1 ^ b2


def _np_uniform_bits(bits):
    f = ((bits >> np.uint32(9)) | np.uint32(0x3F800000)).view(np.float32)
    return np.maximum(np.float32(0.0), f - np.float32(1.0))


def _compute_random_consts_np():
    hi, lo = _iota_2x32(3)
    b1, b2 = _threefry_pair(0, 1, hi, lo)
    k_shuf, k_prefix, k_bern = [(int(b1[i]), int(b2[i])) for i in range(3)]

    u = _np_uniform_bits(_np_random_bits(*k_shuf, N * G)).reshape(N, G)
    shuffle_idx = np.argsort(u, axis=-1, kind="stable")[:, :CONTEXT_LEN]

    pw = np.float32(MAX_PREFIX_LEN) / np.arange(MAX_PREFIX_LEN, dtype=np.float32)
    pw[0] = 1.0
    logits = np.log(pw, dtype=np.float32)
    ug = _np_uniform_bits(_np_random_bits(*k_prefix, N * MAX_PREFIX_LEN)
                          ).reshape(N, MAX_PREFIX_LEN)
    tiny = np.float32(np.finfo(np.float32).tiny)
    ug = np.maximum(tiny, ug * (np.float32(1.0) - tiny) + tiny)
    gumbel = -np.log(-np.log(ug), dtype=np.float32).astype(np.float32)
    prefix = np.argmax(logits[None, :] + gumbel, axis=1).astype(np.int32)

    mw = prefix.astype(np.float32) / np.float32(MAX_PREFIX_LEN + 1)
    ub = _np_uniform_bits(_np_random_bits(*k_bern, N * 3)).reshape(N, 3)
    bern = (ub < mw[:, None]).astype(np.int32)
    return (np.ascontiguousarray(shuffle_idx, dtype=np.int32), prefix, bern)


def _compute_random_consts_jax():
    cpu = jax.devices("cpu")[0]
    with jax.default_device(cpu):
        rkey = jax.random.key(1)
        k_shuf, k_prefix, k_bern = jax.random.split(rkey, 3)

        u = jax.random.uniform(k_shuf, (N, G), dtype=jnp.float32)
        shuffle_idx_nc = np.argsort(np.asarray(u), axis=-1, kind="stable")[:, :CONTEXT_LEN]

        prefix_weights = MAX_PREFIX_LEN / jnp.arange(MAX_PREFIX_LEN, dtype=jnp.float32)
        prefix_weights = prefix_weights.at[0].set(1.0)
        prefix_len_n = jax.random.categorical(k_prefix, jnp.log(prefix_weights), shape=(N,))

        metadata_weights_n = prefix_len_n.astype(jnp.float32) / (MAX_PREFIX_LEN + 1)
        bern_n3 = jax.random.bernoulli(
            k_bern, jnp.broadcast_to(metadata_weights_n[:, None], (N, 3)))

    return (
        np.ascontiguousarray(shuffle_idx_nc, dtype=np.int32),
        np.asarray(prefix_len_n, dtype=np.int32),
        np.asarray(bern_n3, dtype=np.int32),
    )


def _compute_random_consts():
    try:
        return _compute_random_consts_jax()
    except Exception:
        return _compute_random_consts_np()


_RANDOM_CONSTS = _compute_random_consts()


@functools.lru_cache(maxsize=1)
def _random_consts():
    return _RANDOM_CONSTS


_SLAB = 2
_NSLABS = ROWS_PER_WORKER // _SLAB
_C = CONTEXT_LEN


def _sc_gather_body(table_hbm, idx_hbm, outv_hbm,
                    buf0, buf1, ibuf0, ibuf1, obuf0, obuf1,
                    s_in0, s_in1, s_idx0, s_idx1, s_out0, s_out1):
    wid = lax.axis_index("s") * 2 + lax.axis_index("c")
    base = wid * ROWS_PER_WORKER

    pltpu.async_copy(table_hbm.at[pl.ds(base, _SLAB)], buf0, s_in0)
    pltpu.async_copy(idx_hbm.at[pl.ds(base, _SLAB)], ibuf0, s_idx0)

    slots = ((buf0, ibuf0, obuf0, s_in0, s_idx0, s_out0),
             (buf1, ibuf1, obuf1, s_in1, s_idx1, s_out1))

    def outer(k, carry):
        for slot in range(2):
            buf, ibuf, obuf, s_in, s_idx, s_out = slots[slot]
            nbuf, nibuf, _, ns_in, ns_idx, _ = slots[1 - slot]
            slab_row = base + (2 * k + slot) * _SLAB
            if slot == 0:
                pltpu.async_copy(table_hbm.at[pl.ds(slab_row + _SLAB, _SLAB)],
                                 nbuf, ns_in)
                pltpu.async_copy(idx_hbm.at[pl.ds(slab_row + _SLAB, _SLAB)],
                                 nibuf, ns_idx)
            else:
                @pl.when(k < _NSLABS // 2 - 1)
                def _():
                    pltpu.async_copy(
                        table_hbm.at[pl.ds(slab_row + _SLAB, _SLAB)], nbuf, ns_in)
                    pltpu.async_copy(
                        idx_hbm.at[pl.ds(slab_row + _SLAB, _SLAB)], nibuf, ns_idx)
            pltpu.make_async_copy(table_hbm.at[pl.ds(slab_row, _SLAB)], buf, s_in).wait()
            pltpu.make_async_copy(idx_hbm.at[pl.ds(slab_row, _SLAB)], ibuf, s_idx).wait()
            @pl.when(k > 0)
            def _():
                pltpu.make_async_copy(
                    obuf, outv_hbm.at[pl.ds(slab_row, _SLAB)], s_out).wait()
            for r in range(_SLAB):
                row_sel = jnp.full((LANES,), r, jnp.int32)
                for j in range(_C // LANES):
                    ids = ibuf[r, pl.ds(j * LANES, LANES)]
                    obuf[r, pl.ds(j * LANES, LANES)] = plsc.load_gather(
                        buf, [row_sel, ids])
            pltpu.async_copy(obuf, outv_hbm.at[pl.ds(slab_row, _SLAB)], s_out)
        return carry

    lax.fori_loop(0, _NSLABS // 2, outer, 0)

    last0 = base + (_NSLABS - 2) * _SLAB
    last1 = base + (_NSLABS - 1) * _SLAB
    pltpu.make_async_copy(obuf0, outv_hbm.at[pl.ds(last0, _SLAB)], s_out0).wait()
    pltpu.make_async_copy(obuf1, outv_hbm.at[pl.ds(last1, _SLAB)], s_out1).wait()


def _sc_gather(table, idx):
    mesh = plsc.VectorSubcoreMesh(core_axis_name="c", subcore_axis_name="s")
    fn = pl.kernel(
        _sc_gather_body,
        out_type=jax.ShapeDtypeStruct((N, CONTEXT_LEN), jnp.float32),
        mesh=mesh,
        scratch_types=[
            pltpu.VMEM((_SLAB, G), jnp.float32),
            pltpu.VMEM((_SLAB, G), jnp.float32),
            pltpu.VMEM((_SLAB, _C), jnp.int32),
            pltpu.VMEM((_SLAB, _C), jnp.int32),
            pltpu.VMEM((_SLAB, _C), jnp.float32),
            pltpu.VMEM((_SLAB, _C), jnp.float32),
            pltpu.SemaphoreType.DMA,
            pltpu.SemaphoreType.DMA,
            pltpu.SemaphoreType.DMA,
            pltpu.SemaphoreType.DMA,
            pltpu.SemaphoreType.DMA,
            pltpu.SemaphoreType.DMA,
        ],
        compiler_params=pltpu.CompilerParams(needs_layout_passes=False),
    )
    return fn(table, idx)


_ROWS_PER_BLOCK = 32
_W = CONTEXT_LEN + 3


def _asm_dep_body(g_ref, pref_ref, ch0_o, glab_o):
    R = g_ref.shape[0]
    C = CONTEXT_LEN
    lane = lax.broadcasted_iota(jnp.int32, (R, C), 1)
    p = pref_ref[...]
    suffix = lane >= p
    g = g_ref[...]
    lg = jnp.log1p(g)
    ch0_o[...] = jnp.where(suffix, 0.0, lg)
    glab_o[:, :C] = jnp.clip(g, 0, N_GENE_VALUES - 1).astype(jnp.int32)
    glab_o[:, C:] = jnp.zeros((R, 3), jnp.int32)


def _make_asm_dep(interpret=False):
    R = _ROWS_PER_BLOCK
    C = CONTEXT_LEN
    return pl.pallas_call(
        _asm_dep_body,
        grid=(N // R,),
        in_specs=[pl.BlockSpec((R, C), lambda i: (i, 0)),
                  pl.BlockSpec((R, 1), lambda i: (i, 0))],
        out_specs=[pl.BlockSpec((R, C), lambda i: (i, 0)),
                   pl.BlockSpec((R, _W), lambda i: (i, 0))],
        out_shape=[
            jax.ShapeDtypeStruct((N, C), jnp.float32),
            jax.ShapeDtypeStruct((N, _W), jnp.int32),
        ],
        interpret=interpret,
    )


def _asm_body(pref_ref, umis_ref, assay_ref, susp_ref,
              ch1_o, ch2_o, assay_o, susp_o):
    R = pref_ref.shape[0]
    C = CONTEXT_LEN
    lane = lax.broadcasted_iota(jnp.int32, (R, C), 1)
    p = pref_ref[...]
    suffix = lane >= p
    ch1_o[...] = suffix.astype(jnp.float32)
    ch2_o[...] = jnp.broadcast_to(jnp.log1p(umis_ref[...]), (R, C))
    assay_o[...] = jnp.broadcast_to(assay_ref[...], (R, C))
    susp_o[...] = jnp.broadcast_to(susp_ref[...], (R, C))


def _make_asm(interpret=False):
    R = _ROWS_PER_BLOCK
    C = CONTEXT_LEN
    row_spec_c = pl.BlockSpec((R, C), lambda i: (i, 0))
    col1 = pl.BlockSpec((R, 1), lambda i: (i, 0))
    return pl.pallas_call(
        _asm_body,
        grid=(N // R,),
        in_specs=[col1, col1, col1, col1],
        out_specs=[row_spec_c, row_spec_c, row_spec_c, row_spec_c],
        out_shape=[
            jax.ShapeDtypeStruct((N, C), jnp.float32),
            jax.ShapeDtypeStruct((N, C), jnp.float32),
            jax.ShapeDtypeStruct((N, C), jnp.int32),
            jax.ShapeDtypeStruct((N, C), jnp.int32),
        ],
        interpret=interpret,
    )


_NB = 128


def _asm_t_body(pref_ref, ct_ref, ds_ref, sex_ref, bern_ref,
                prompt_o, ctlab_o, dslab_o, sexlab_o,
                gw_o, ctw_o, dsw_o, sexw_o):
    NB = pref_ref.shape[1]
    W = _W
    C = CONTEXT_LEN
    row = lax.broadcasted_iota(jnp.int32, (W, NB), 0)
    p = pref_ref[...]
    ct = ct_ref[...]
    ds = ds_ref[...]
    sx = sex_ref[...]
    bern = bern_ref[...]
    b0, b1, b2 = bern[0:1, :], bern[1:2, :], bern[2:3, :]
    m0 = (ct < 0).astype(jnp.int32)
    m1 = (ds < 0).astype(jnp.int32)
    m2 = (sx < 0).astype(jnp.int32)
    q0, q1, q2 = b0 * m0, b1 * m1, b2 * m2
    pm0, pm1, pm2 = (1 - b0) * m0, (1 - b1) * m1, (1 - b2) * m2

    is_gene = row < C
    r_ct = row == C
    r_ds = row == C + 1
    r_sx = row == C + 2
    suffix = is_gene & (row >= p)

    prompt_i = jnp.where(is_gene, (row < p).astype(jnp.int32),
                         jnp.where(r_ct, pm0, jnp.where(r_ds, pm1, pm2)))
    prompt_o[...] = prompt_i != 0
    zi = jnp.zeros((W, NB), jnp.int32)
    ctlab_o[...] = jnp.where(r_ct, jnp.maximum(ct, 0), zi)
    dslab_o[...] = jnp.where(r_ds, jnp.maximum(ds, 0), zi)
    sexlab_o[...] = jnp.where(r_sx, jnp.maximum(sx, 0), zi)

    s = (C - p).astype(jnp.float32)
    winv = 1.0 / s
    zf = jnp.zeros((W, NB), jnp.float32)
    gw_o[...] = jnp.where(suffix, winv, zf)
    ctw_o[...] = jnp.where(r_ct, q0.astype(jnp.float32), zf)
    dsw_o[...] = jnp.where(r_ds, q1.astype(jnp.float32), zf)
    sexw_o[...] = jnp.where(r_sx, q2.astype(jnp.float32), zf)


def _make_asm_t(interpret=False):
    wspec = pl.BlockSpec((_W, _NB), lambda i: (0, i))
    spec1 = pl.BlockSpec((1, _NB), lambda i: (0, i))
    spec3 = pl.BlockSpec((3, _NB), lambda i: (0, i))
    return pl.pallas_call(
        _asm_t_body,
        grid=(N // _NB,),
        in_specs=[spec1, spec1, spec1, spec1, spec3],
        out_specs=[wspec] * 8,
        out_shape=[
            jax.ShapeDtypeStruct((_W, N), jnp.bool_),
            jax.ShapeDtypeStruct((_W, N), jnp.int32),
            jax.ShapeDtypeStruct((_W, N), jnp.int32),
            jax.ShapeDtypeStruct((_W, N), jnp.int32),
            jax.ShapeDtypeStruct((_W, N), jnp.float32),
            jax.ShapeDtypeStruct((_W, N), jnp.float32),
            jax.ShapeDtypeStruct((_W, N), jnp.float32),
            jax.ShapeDtypeStruct((_W, N), jnp.float32),
        ],
        interpret=interpret,
    )


def kernel(cell_type_n, sex_n, development_stage_n, gene_value_ng,
           total_mrna_umis_n, assay_n, suspension_type_n):
    idx_np, prefix_np, bern_np = _random_consts()
    idx_c = jnp.asarray(idx_np)
    prefix_c = jnp.asarray(prefix_np).reshape(N, 1)
    bern_c = jnp.asarray(bern_np)

    gathered = _sc_gather(gene_value_ng, idx_c)
    gene_id_nc = idx_c

    ch1, ch2, assay_nc, susp_nc = _make_asm()(
        prefix_c,
        total_mrna_umis_n.reshape(N, 1).astype(jnp.float32),
        assay_n.reshape(N, 1).astype(jnp.int32),
        suspension_type_n.reshape(N, 1).astype(jnp.int32),
    )

    (promptT, ctlabT, dslabT, sexlabT, gwT, ctwT, dswT, sexwT) = _make_asm_t()(
        jnp.asarray(prefix_np).reshape(1, N),
        cell_type_n.reshape(1, N).astype(jnp.int32),
        development_stage_n.reshape(1, N).astype(jnp.int32),
        sex_n.reshape(1, N).astype(jnp.int32),
        jnp.asarray(bern_np.T),
    )
    prompt_mask = promptT.T
    ctlab = ctlabT.T
    dslab = dslabT.T
    sexlab = sexlabT.T
    gw = gwT.T
    ctw = ctwT.T
    dsw = dswT.T
    sexw = sexwT.T

    ch0, glab = _make_asm_dep()(gathered, prefix_c)

    gene_value_nc3 = jnp.stack([ch0, ch1, ch2], axis=2)

    measured = jnp.stack([cell_type_n < 0, development_stage_n < 0, sex_n < 0], axis=1)
    q3 = (bern_c != 0) & measured
    cell_type_tok = jnp.where(q3[:, 0], N_CELL_TYPES,
                              jnp.maximum(cell_type_n, 0)).astype(jnp.int32)
    development_stage_tok = jnp.where(q3[:, 1], N_DEV_STAGES,
                                      jnp.maximum(development_stage_n, 0)).astype(jnp.int32)
    sex_tok = jnp.where(q3[:, 2], N_SEXES,
                        jnp.maximum(sex_n, 0)).astype(jnp.int32)

    return (
        gene_id_nc,
        gene_value_nc3,
        assay_nc,
        susp_nc,
        cell_type_tok,
        sex_tok,
        development_stage_tok,
        prompt_mask,
        glab,
        ctlab,
        dslab,
        sexlab,
        gw,
        ctw,
        dsw,
        sexw,
    )


# speedup vs baseline: 1.9490x; 1.0755x over previous
import functools

import jax
import jax.numpy as jnp
import numpy as np
from jax import lax
from jax.experimental import pallas as pl
from jax.experimental.pallas import tpu as pltpu
from jax.experimental.pallas import tpu_sc as plsc

MAX_PREFIX_LEN = 512
CONTEXT_LEN = 2048
N_GENE_VALUES = 2048
N_CELL_TYPES = 200
N_DEV_STAGES = 50
N_SEXES = 2
N = 1024
G = 19264

NUM_WORKERS = 32
ROWS_PER_WORKER = N // NUM_WORKERS
LANES = 16


def _rotl(x, r):
    return ((x << np.uint32(r)) | (x >> np.uint32(32 - r))).astype(np.uint32)


def _threefry_pair(k1, k2, x0, x1):
    x0 = x0.astype(np.uint32).copy()
    x1 = x1.astype(np.uint32).copy()
    ks = [np.uint32(k1), np.uint32(k2),
          np.uint32(np.uint32(k1) ^ np.uint32(k2) ^ np.uint32(0x1BD11BDA))]
    rotations = [[13, 15, 26, 6], [17, 29, 16, 24]]
    with np.errstate(over="ignore"):
        x0 = (x0 + ks[0]).astype(np.uint32)
        x1 = (x1 + ks[1]).astype(np.uint32)
        for i in range(5):
            for r in rotations[i % 2]:
                x0 = (x0 + x1).astype(np.uint32)
                x1 = _rotl(x1, r) ^ x0
            x0 = (x0 + ks[(i + 1) % 3]).astype(np.uint32)
            x1 = (x1 + ks[(i + 2) % 3] + np.uint32(i + 1)).astype(np.uint32)
    return x0, x1


def _iota_2x32(n):
    i = np.arange(n, dtype=np.uint64)
    return ((i >> np.uint64(32)).astype(np.uint32),
            (i & np.uint64(0xFFFFFFFF)).astype(np.uint32))


def _np_random_bits(k1, k2, n):
    hi, lo = _iota_2x32(n)
    b1, b2 = _threefry_pair(k1, k2, hi, lo)
    return b1 ^ b2


def _np_uniform_bits(bits):
    f = ((bits >> np.uint32(9)) | np.uint32(0x3F800000)).view(np.float32)
    return np.maximum(np.float32(0.0), f - np.float32(1.0))


def _compute_random_consts_np():
    hi, lo = _iota_2x32(3)
    b1, b2 = _threefry_pair(0, 1, hi, lo)
    k_shuf, k_prefix, k_bern = [(int(b1[i]), int(b2[i])) for i in range(3)]

    u = _np_uniform_bits(_np_random_bits(*k_shuf, N * G)).reshape(N, G)
    shuffle_idx = np.argsort(u, axis=-1, kind="stable")[:, :CONTEXT_LEN]

    pw = np.float32(MAX_PREFIX_LEN) / np.arange(MAX_PREFIX_LEN, dtype=np.float32)
    pw[0] = 1.0
    logits = np.log(pw, dtype=np.float32)
    ug = _np_uniform_bits(_np_random_bits(*k_prefix, N * MAX_PREFIX_LEN)
                          ).reshape(N, MAX_PREFIX_LEN)
    tiny = np.float32(np.finfo(np.float32).tiny)
    ug = np.maximum(tiny, ug * (np.float32(1.0) - tiny) + tiny)
    gumbel = -np.log(-np.log(ug), dtype=np.float32).astype(np.float32)
    prefix = np.argmax(logits[None, :] + gumbel, axis=1).astype(np.int32)

    mw = prefix.astype(np.float32) / np.float32(MAX_PREFIX_LEN + 1)
    ub = _np_uniform_bits(_np_random_bits(*k_bern, N * 3)).reshape(N, 3)
    bern = (ub < mw[:, None]).astype(np.int32)
    return (np.ascontiguousarray(shuffle_idx, dtype=np.int32), prefix, bern)


def _compute_random_consts_jax():
    cpu = jax.devices("cpu")[0]
    with jax.default_device(cpu):
        rkey = jax.random.key(1)
        k_shuf, k_prefix, k_bern = jax.random.split(rkey, 3)

        u = jax.random.uniform(k_shuf, (N, G), dtype=jnp.float32)
        shuffle_idx_nc = np.argsort(np.asarray(u), axis=-1, kind="stable")[:, :CONTEXT_LEN]

        prefix_weights = MAX_PREFIX_LEN / jnp.arange(MAX_PREFIX_LEN, dtype=jnp.float32)
        prefix_weights = prefix_weights.at[0].set(1.0)
        prefix_len_n = jax.random.categorical(k_prefix, jnp.log(prefix_weights), shape=(N,))

        metadata_weights_n = prefix_len_n.astype(jnp.float32) / (MAX_PREFIX_LEN + 1)
        bern_n3 = jax.random.bernoulli(
            k_bern, jnp.broadcast_to(metadata_weights_n[:, None], (N, 3)))

    return (
        np.ascontiguousarray(shuffle_idx_nc, dtype=np.int32),
        np.asarray(prefix_len_n, dtype=np.int32),
        np.asarray(bern_n3, dtype=np.int32),
    )


def _compute_random_consts():
    try:
        return _compute_random_consts_jax()
    except Exception:
        return _compute_random_consts_np()


_RANDOM_CONSTS = _compute_random_consts()


@functools.lru_cache(maxsize=1)
def _random_consts():
    return _RANDOM_CONSTS


_SLAB = 2
_NSLABS = ROWS_PER_WORKER // _SLAB
_C = CONTEXT_LEN


def _sc_gather_body(table_hbm, idx_hbm, outv_hbm,
                    buf0, buf1, ibuf0, ibuf1, obuf0, obuf1,
                    s_in0, s_in1, s_idx0, s_idx1, s_out0, s_out1):
    wid = lax.axis_index("s") * 2 + lax.axis_index("c")
    base = wid * ROWS_PER_WORKER

    pltpu.async_copy(table_hbm.at[pl.ds(base, _SLAB)], buf0, s_in0)
    pltpu.async_copy(idx_hbm.at[pl.ds(base, _SLAB)], ibuf0, s_idx0)

    slots = ((buf0, ibuf0, obuf0, s_in0, s_idx0, s_out0),
             (buf1, ibuf1, obuf1, s_in1, s_idx1, s_out1))

    def outer(k, carry):
        for slot in range(2):
            buf, ibuf, obuf, s_in, s_idx, s_out = slots[slot]
            nbuf, nibuf, _, ns_in, ns_idx, _ = slots[1 - slot]
            slab_row = base + (2 * k + slot) * _SLAB
            if slot == 0:
                pltpu.async_copy(table_hbm.at[pl.ds(slab_row + _SLAB, _SLAB)],
                                 nbuf, ns_in)
                pltpu.async_copy(idx_hbm.at[pl.ds(slab_row + _SLAB, _SLAB)],
                                 nibuf, ns_idx)
            else:
                @pl.when(k < _NSLABS // 2 - 1)
                def _():
                    pltpu.async_copy(
                        table_hbm.at[pl.ds(slab_row + _SLAB, _SLAB)], nbuf, ns_in)
                    pltpu.async_copy(
                        idx_hbm.at[pl.ds(slab_row + _SLAB, _SLAB)], nibuf, ns_idx)
            pltpu.make_async_copy(table_hbm.at[pl.ds(slab_row, _SLAB)], buf, s_in).wait()
            pltpu.make_async_copy(idx_hbm.at[pl.ds(slab_row, _SLAB)], ibuf, s_idx).wait()
            @pl.when(k > 0)
            def _():
                pltpu.make_async_copy(
                    obuf, outv_hbm.at[pl.ds(slab_row, _SLAB)], s_out).wait()
            for r in range(_SLAB):
                row_sel = jnp.full((LANES,), r, jnp.int32)
                for j in range(_C // LANES):
                    ids = ibuf[r, pl.ds(j * LANES, LANES)]
                    obuf[r, pl.ds(j * LANES, LANES)] = plsc.load_gather(
                        buf, [row_sel, ids])
            pltpu.async_copy(obuf, outv_hbm.at[pl.ds(slab_row, _SLAB)], s_out)
        return carry

    lax.fori_loop(0, _NSLABS // 2, outer, 0)

    last0 = base + (_NSLABS - 2) * _SLAB
    last1 = base + (_NSLABS - 1) * _SLAB
    pltpu.make_async_copy(obuf0, outv_hbm.at[pl.ds(last0, _SLAB)], s_out0).wait()
    pltpu.make_async_copy(obuf1, outv_hbm.at[pl.ds(last1, _SLAB)], s_out1).wait()


def _sc_gather(table, idx):
    mesh = plsc.VectorSubcoreMesh(core_axis_name="c", subcore_axis_name="s")
    fn = pl.kernel(
        _sc_gather_body,
        out_type=jax.ShapeDtypeStruct((N, CONTEXT_LEN), jnp.float32),
        mesh=mesh,
        scratch_types=[
            pltpu.VMEM((_SLAB, G), jnp.float32),
            pltpu.VMEM((_SLAB, G), jnp.float32),
            pltpu.VMEM((_SLAB, _C), jnp.int32),
            pltpu.VMEM((_SLAB, _C), jnp.int32),
            pltpu.VMEM((_SLAB, _C), jnp.float32),
            pltpu.VMEM((_SLAB, _C), jnp.float32),
            pltpu.SemaphoreType.DMA,
            pltpu.SemaphoreType.DMA,
            pltpu.SemaphoreType.DMA,
            pltpu.SemaphoreType.DMA,
            pltpu.SemaphoreType.DMA,
            pltpu.SemaphoreType.DMA,
        ],
        compiler_params=pltpu.CompilerParams(needs_layout_passes=False),
    )
    return fn(table, idx)


_ROWS_PER_BLOCK = 32
_W = CONTEXT_LEN + 3


def _asm_dep_body(g_ref, pref_ref, umis_ref, nc3_o, glab_o):
    R = g_ref.shape[0]
    C = CONTEXT_LEN
    lane = lax.broadcasted_iota(jnp.int32, (R, C), 1)
    p = pref_ref[...]
    suffix = lane >= p
    g = g_ref[...]
    lg = jnp.log1p(g)
    nc3_o[0] = jnp.where(suffix, 0.0, lg)
    nc3_o[1] = suffix.astype(jnp.float32)
    nc3_o[2] = jnp.broadcast_to(jnp.log1p(umis_ref[...]), (R, C))
    glab_o[:, :C] = jnp.clip(g, 0, N_GENE_VALUES - 1).astype(jnp.int32)
    glab_o[:, C:] = jnp.zeros((R, 3), jnp.int32)


def _make_asm_dep(interpret=False):
    R = _ROWS_PER_BLOCK
    C = CONTEXT_LEN
    return pl.pallas_call(
        _asm_dep_body,
        grid=(N // R,),
        in_specs=[pl.BlockSpec((R, C), lambda i: (i, 0)),
                  pl.BlockSpec((R, 1), lambda i: (i, 0)),
                  pl.BlockSpec((R, 1), lambda i: (i, 0))],
        out_specs=[pl.BlockSpec((3, R, C), lambda i: (0, i, 0)),
                   pl.BlockSpec((R, _W), lambda i: (i, 0))],
        out_shape=[
            jax.ShapeDtypeStruct((3, N, C), jnp.float32),
            jax.ShapeDtypeStruct((N, _W), jnp.int32),
        ],
        interpret=interpret,
    )


def _asm_body(assay_ref, susp_ref, assay_o, susp_o):
    R = assay_ref.shape[0]
    C = CONTEXT_LEN
    assay_o[...] = jnp.broadcast_to(assay_ref[...], (R, C))
    susp_o[...] = jnp.broadcast_to(susp_ref[...], (R, C))


def _make_asm(interpret=False):
    R = _ROWS_PER_BLOCK
    C = CONTEXT_LEN
    row_spec_c = pl.BlockSpec((R, C), lambda i: (i, 0))
    col1 = pl.BlockSpec((R, 1), lambda i: (i, 0))
    return pl.pallas_call(
        _asm_body,
        grid=(N // R,),
        in_specs=[col1, col1],
        out_specs=[row_spec_c, row_spec_c],
        out_shape=[
            jax.ShapeDtypeStruct((N, C), jnp.int32),
            jax.ShapeDtypeStruct((N, C), jnp.int32),
        ],
        interpret=interpret,
    )


_NB = 128


def _asm_t_body(pref_ref, ct_ref, ds_ref, sex_ref, bern_ref,
                prompt_o, ctlab_o, dslab_o, sexlab_o,
                gw_o, ctw_o, dsw_o, sexw_o):
    NB = pref_ref.shape[1]
    W = _W
    C = CONTEXT_LEN
    row = lax.broadcasted_iota(jnp.int32, (W, NB), 0)
    p = pref_ref[...]
    ct = ct_ref[...]
    ds = ds_ref[...]
    sx = sex_ref[...]
    bern = bern_ref[...]
    b0, b1, b2 = bern[0:1, :], bern[1:2, :], bern[2:3, :]
    m0 = (ct < 0).astype(jnp.int32)
    m1 = (ds < 0).astype(jnp.int32)
    m2 = (sx < 0).astype(jnp.int32)
    q0, q1, q2 = b0 * m0, b1 * m1, b2 * m2
    pm0, pm1, pm2 = (1 - b0) * m0, (1 - b1) * m1, (1 - b2) * m2

    is_gene = row < C
    r_ct = row == C
    r_ds = row == C + 1
    r_sx = row == C + 2
    suffix = is_gene & (row >= p)

    prompt_i = jnp.where(is_gene, (row < p).astype(jnp.int32),
                         jnp.where(r_ct, pm0, jnp.where(r_ds, pm1, pm2)))
    prompt_o[...] = prompt_i != 0
    zi = jnp.zeros((W, NB), jnp.int32)
    ctlab_o[...] = jnp.where(r_ct, jnp.maximum(ct, 0), zi)
    dslab_o[...] = jnp.where(r_ds, jnp.maximum(ds, 0), zi)
    sexlab_o[...] = jnp.where(r_sx, jnp.maximum(sx, 0), zi)

    s = (C - p).astype(jnp.float32)
    winv = 1.0 / s
    zf = jnp.zeros((W, NB), jnp.float32)
    gw_o[...] = jnp.where(suffix, winv, zf)
    ctw_o[...] = jnp.where(r_ct, q0.astype(jnp.float32), zf)
    dsw_o[...] = jnp.where(r_ds, q1.astype(jnp.float32), zf)
    sexw_o[...] = jnp.where(r_sx, q2.astype(jnp.float32), zf)


def _make_asm_t(interpret=False):
    wspec = pl.BlockSpec((_W, _NB), lambda i: (0, i))
    spec1 = pl.BlockSpec((1, _NB), lambda i: (0, i))
    spec3 = pl.BlockSpec((3, _NB), lambda i: (0, i))
    return pl.pallas_call(
        _asm_t_body,
        grid=(N // _NB,),
        in_specs=[spec1, spec1, spec1, spec1, spec3],
        out_specs=[wspec] * 8,
        out_shape=[
            jax.ShapeDtypeStruct((_W, N), jnp.bool_),
            jax.ShapeDtypeStruct((_W, N), jnp.int32),
            jax.ShapeDtypeStruct((_W, N), jnp.int32),
            jax.ShapeDtypeStruct((_W, N), jnp.int32),
            jax.ShapeDtypeStruct((_W, N), jnp.float32),
            jax.ShapeDtypeStruct((_W, N), jnp.float32),
            jax.ShapeDtypeStruct((_W, N), jnp.float32),
            jax.ShapeDtypeStruct((_W, N), jnp.float32),
        ],
        interpret=interpret,
    )


def kernel(cell_type_n, sex_n, development_stage_n, gene_value_ng,
           total_mrna_umis_n, assay_n, suspension_type_n):
    idx_np, prefix_np, bern_np = _random_consts()
    idx_c = jnp.asarray(idx_np)
    prefix_c = jnp.asarray(prefix_np).reshape(N, 1)
    bern_c = jnp.asarray(bern_np)

    gathered = _sc_gather(gene_value_ng, idx_c)
    gene_id_nc = idx_c

    assay_nc, susp_nc = _make_asm()(
        assay_n.reshape(N, 1).astype(jnp.int32),
        suspension_type_n.reshape(N, 1).astype(jnp.int32),
    )

    (promptT, ctlabT, dslabT, sexlabT, gwT, ctwT, dswT, sexwT) = _make_asm_t()(
        jnp.asarray(prefix_np).reshape(1, N),
        cell_type_n.reshape(1, N).astype(jnp.int32),
        development_stage_n.reshape(1, N).astype(jnp.int32),
        sex_n.reshape(1, N).astype(jnp.int32),
        jnp.asarray(bern_np.T),
    )
    prompt_mask = promptT.T
    ctlab = ctlabT.T
    dslab = dslabT.T
    sexlab = sexlabT.T
    gw = gwT.T
    ctw = ctwT.T
    dsw = dswT.T
    sexw = sexwT.T

    nc3p, glab = _make_asm_dep()(
        gathered, prefix_c,
        total_mrna_umis_n.reshape(N, 1).astype(jnp.float32))

    gene_value_nc3 = jnp.moveaxis(nc3p, 0, 2)

    measured = jnp.stack([cell_type_n < 0, development_stage_n < 0, sex_n < 0], axis=1)
    q3 = (bern_c != 0) & measured
    cell_type_tok = jnp.where(q3[:, 0], N_CELL_TYPES,
                              jnp.maximum(cell_type_n, 0)).astype(jnp.int32)
    development_stage_tok = jnp.where(q3[:, 1], N_DEV_STAGES,
                                      jnp.maximum(development_stage_n, 0)).astype(jnp.int32)
    sex_tok = jnp.where(q3[:, 2], N_SEXES,
                        jnp.maximum(sex_n, 0)).astype(jnp.int32)

    return (
        gene_id_nc,
        gene_value_nc3,
        assay_nc,
        susp_nc,
        cell_type_tok,
        sex_tok,
        development_stage_tok,
        prompt_mask,
        glab,
        ctlab,
        dslab,
        sexlab,
        gw,
        ctw,
        dsw,
        sexw,
    )
